# baseline (speedup 1.0000x reference)
import functools

import jax
import jax.numpy as jnp
from jax import lax
from jax.experimental import pallas as pl
from jax.experimental.pallas import tpu as pltpu
from jax.experimental.pallas import tpu_sc as plsc

V = 1_000_000
D = 32
B = 16384
NC = 2
NS = 16
NW = NC * NS
CH = 4
CK = B // (NW * CH)


def _glove_gather(words2d, ctx2d, w_emb, w_bias, c_emb, c_bias):
  mesh = plsc.VectorSubcoreMesh(core_axis_name="c", subcore_axis_name="s")

  @functools.partial(
      pl.kernel,
      mesh=mesh,
      compiler_params=pltpu.CompilerParams(use_tc_tiling_on_sc=False),
      out_type=(
          jax.ShapeDtypeStruct((NW * CH, CK, D), jnp.float32),
          jax.ShapeDtypeStruct((NW * CH, CK), jnp.float32),
          jax.ShapeDtypeStruct((NW * CH, CK, D), jnp.float32),
          jax.ShapeDtypeStruct((NW * CH, CK), jnp.float32),
      ),
      scratch_types=[
          pltpu.VMEM((CH, CK), jnp.int32),
          pltpu.VMEM((CH, CK), jnp.int32),
          pltpu.VMEM((CH, CK, D), jnp.float32),
          pltpu.VMEM((CH, CK), jnp.float32),
          pltpu.VMEM((CH, CK, D), jnp.float32),
          pltpu.VMEM((CH, CK), jnp.float32),
          pltpu.SemaphoreType.DMA,
          pltpu.SemaphoreType.DMA,
      ],
  )
  def k(words_h, ctx_h, we_h, wb_h, ce_h, cb_h,
        owe_h, owb_h, oce_h, ocb_h,
        widx_v, cidx_v, we_v, wb_v, ce_v, cb_v, gsem, osem):
    wid = lax.axis_index("s") * NC + lax.axis_index("c")
    row0 = wid * CH
    pltpu.sync_copy(words_h.at[pl.ds(row0, CH)], widx_v)
    pltpu.sync_copy(ctx_h.at[pl.ds(row0, CH)], cidx_v)
    gathers = []
    for j in range(CH):
      gathers.append(pltpu.async_copy(we_h.at[widx_v.at[j]], we_v.at[j], gsem))
      gathers.append(pltpu.async_copy(wb_h.at[widx_v.at[j]], wb_v.at[j], gsem))
      gathers.append(pltpu.async_copy(ce_h.at[cidx_v.at[j]], ce_v.at[j], gsem))
      gathers.append(pltpu.async_copy(cb_h.at[cidx_v.at[j]], cb_v.at[j], gsem))
    for g in gathers:
      g.wait()
    outs = [
        pltpu.async_copy(we_v, owe_h.at[pl.ds(row0, CH)], osem),
        pltpu.async_copy(wb_v, owb_h.at[pl.ds(row0, CH)], osem),
        pltpu.async_copy(ce_v, oce_h.at[pl.ds(row0, CH)], osem),
        pltpu.async_copy(cb_v, ocb_h.at[pl.ds(row0, CH)], osem),
    ]
    for o in outs:
      o.wait()

  return k(words2d, ctx2d, w_emb, w_bias, c_emb, c_bias)


def kernel(words, contexts, w_embeddings, w_biases, c_embeddings, c_biases):
  words2d = words.astype(jnp.int32).reshape(NW * CH, CK)
  ctx2d = contexts.astype(jnp.int32).reshape(NW * CH, CK)
  owe, owb, oce, ocb = _glove_gather(
      words2d, ctx2d, w_embeddings, w_biases.reshape(V),
      c_embeddings, c_biases.reshape(V))
  return (owe.reshape(B, D), owb.reshape(B, 1),
          oce.reshape(B, D), ocb.reshape(B, 1))
